# Initial kernel scaffold; baseline (speedup 1.0000x reference)
import jax
import jax.numpy as jnp
from jax.experimental import pallas as pl


def kernel(atomic_numbers, radial_basis, emb_table, W, b):
    raise NotImplementedError("write your pallas kernel here")


# speedup vs baseline: 1.5094x; 1.5094x over previous
import functools

import jax
import jax.numpy as jnp
from jax import lax
from jax.experimental import pallas as pl
from jax.experimental.pallas import tpu as pltpu
from jax.experimental.pallas import tpu_sc as plsc

N_NODES = 10000
N_EDGES = 320000
NUM_ELEMENTS = 100
HIDDEN = 128
NUM_RADIAL = 16

EDGE_BLOCK = 8000

NC = 2
NS = 16
NW = NC * NS
N_PAD = 10240
B_PER_W = N_PAD // NW
CHUNK = 80
N_CHUNKS = B_PER_W // CHUNK


def _proj_kernel(rb_ref, w_ref, b_ref, s_ref, t_ref, m_ref):
    y = jnp.dot(rb_ref[...], w_ref[...], preferred_element_type=jnp.float32)
    y = y + b_ref[...]
    s_ref[...] = y[:, :HIDDEN]
    t_ref[...] = y[:, HIDDEN:2 * HIDDEN]
    m_ref[...] = y[:, 2 * HIDDEN:]


@functools.partial(
    pl.kernel,
    mesh=plsc.VectorSubcoreMesh(core_axis_name="c", subcore_axis_name="s"),
    out_type=jax.ShapeDtypeStruct((N_PAD, HIDDEN), jnp.float32),
    scratch_types=[
        pltpu.VMEM((B_PER_W,), jnp.int32),
        pltpu.VMEM((B_PER_W, HIDDEN), jnp.float32),
        pltpu.SemaphoreType.DMA,
    ],
)
def _sc_gather(idx_hbm, table_hbm, out_hbm, idx_v, rows_v, sem):
    wid = lax.axis_index("s") * NC + lax.axis_index("c")
    base = wid * B_PER_W
    pltpu.sync_copy(idx_hbm.at[pl.ds(base, B_PER_W)], idx_v)
    copies = [
        pltpu.async_copy(
            table_hbm.at[idx_v.at[pl.ds(j * CHUNK, CHUNK)]],
            rows_v.at[pl.ds(j * CHUNK, CHUNK)],
            sem,
        )
        for j in range(N_CHUNKS)
    ]
    for c in copies:
        c.wait()
    pltpu.sync_copy(rows_v, out_hbm.at[pl.ds(base, B_PER_W)])


def kernel(atomic_numbers, radial_basis, emb_table, W, b):
    b2 = b.reshape(1, HIDDEN * 3)

    grid_e = N_EDGES // EDGE_BLOCK
    out_block = pl.BlockSpec((EDGE_BLOCK, HIDDEN), lambda i: (i, 0))
    s, t, m = pl.pallas_call(
        _proj_kernel,
        grid=(grid_e,),
        in_specs=[
            pl.BlockSpec((EDGE_BLOCK, NUM_RADIAL), lambda i: (i, 0)),
            pl.BlockSpec((NUM_RADIAL, HIDDEN * 3), lambda i: (0, 0)),
            pl.BlockSpec((1, HIDDEN * 3), lambda i: (0, 0)),
        ],
        out_specs=[out_block, out_block, out_block],
        out_shape=[jax.ShapeDtypeStruct((N_EDGES, HIDDEN), jnp.float32)] * 3,
    )(radial_basis, W, b2)

    idx = jnp.pad(atomic_numbers.astype(jnp.int32) - 1, (0, N_PAD - N_NODES))
    h_pad = _sc_gather(idx, emb_table)
    h = h_pad[:N_NODES]

    return (h, m, s, t)


# speedup vs baseline: 1.5109x; 1.0011x over previous
import functools

import jax
import jax.numpy as jnp
from jax import lax
from jax.experimental import pallas as pl
from jax.experimental.pallas import tpu as pltpu
from jax.experimental.pallas import tpu_sc as plsc

N_NODES = 10000
N_EDGES = 320000
NUM_ELEMENTS = 100
HIDDEN = 128
NUM_RADIAL = 16

EDGE_BLOCK = 8000

NC = 2
NS = 16
NW = NC * NS
N_PAD = 10240
B_PER_W = N_PAD // NW
CHUNK = 80
N_CHUNKS = B_PER_W // CHUNK


def _proj_kernel(rb_ref, w_ref, b_ref, s_ref, t_ref, m_ref):
    y = jnp.dot(rb_ref[...], w_ref[...], preferred_element_type=jnp.float32)
    y = y + b_ref[...]
    s_ref[...] = y[:, :HIDDEN]
    t_ref[...] = y[:, HIDDEN:2 * HIDDEN]
    m_ref[...] = y[:, 2 * HIDDEN:]


@functools.partial(
    pl.kernel,
    mesh=plsc.VectorSubcoreMesh(core_axis_name="c", subcore_axis_name="s"),
    out_type=jax.ShapeDtypeStruct((N_PAD, HIDDEN), jnp.float32),
    scratch_types=[
        pltpu.VMEM((B_PER_W,), jnp.int32),
        pltpu.VMEM((B_PER_W, HIDDEN), jnp.float32),
        pltpu.SemaphoreType.DMA,
    ],
)
def _sc_gather(idx_hbm, table_hbm, out_hbm, idx_v, rows_v, sem):
    wid = lax.axis_index("s") * NC + lax.axis_index("c")
    base = wid * B_PER_W
    pltpu.sync_copy(idx_hbm.at[pl.ds(base, B_PER_W)], idx_v)
    copies = [
        pltpu.async_copy(
            table_hbm.at[idx_v.at[pl.ds(j * CHUNK, CHUNK)]],
            rows_v.at[pl.ds(j * CHUNK, CHUNK)],
            sem,
        )
        for j in range(N_CHUNKS)
    ]
    for c in copies:
        c.wait()
    pltpu.sync_copy(rows_v, out_hbm.at[pl.ds(base, B_PER_W)])


def kernel(atomic_numbers, radial_basis, emb_table, W, b):
    idx = jnp.pad(atomic_numbers.astype(jnp.int32) - 1, (0, N_PAD - N_NODES))
    h_pad = _sc_gather(idx, emb_table)
    h = h_pad[:N_NODES]

    b2 = b.reshape(1, HIDDEN * 3)

    grid_e = N_EDGES // EDGE_BLOCK
    out_block = pl.BlockSpec((EDGE_BLOCK, HIDDEN), lambda i: (i, 0))
    s, t, m = pl.pallas_call(
        _proj_kernel,
        grid=(grid_e,),
        in_specs=[
            pl.BlockSpec((EDGE_BLOCK, NUM_RADIAL), lambda i: (i, 0)),
            pl.BlockSpec((NUM_RADIAL, HIDDEN * 3), lambda i: (0, 0)),
            pl.BlockSpec((1, HIDDEN * 3), lambda i: (0, 0)),
        ],
        out_specs=[out_block, out_block, out_block],
        out_shape=[jax.ShapeDtypeStruct((N_EDGES, HIDDEN), jnp.float32)] * 3,
    )(radial_basis, W, b2)

    return (h, m, s, t)


# speedup vs baseline: 1.5494x; 1.0254x over previous
import functools

import jax
import jax.numpy as jnp
from jax import lax
from jax.experimental import pallas as pl
from jax.experimental.pallas import tpu as pltpu
from jax.experimental.pallas import tpu_sc as plsc

N_NODES = 10000
N_EDGES = 320000
NUM_ELEMENTS = 100
HIDDEN = 128
NUM_RADIAL = 16

EDGE_BLOCK = 8000

NC = 2
NS = 16
NW = NC * NS
CHUNK = 80
TOTAL_CHUNKS = N_NODES // CHUNK
MAX_ROUNDS = -(-TOTAL_CHUNKS // NW)


def _proj_kernel(rb_ref, w_ref, b_ref, s_ref, t_ref, m_ref):
    y = jnp.dot(rb_ref[...], w_ref[...], preferred_element_type=jnp.float32)
    y = y + b_ref[...]
    s_ref[...] = y[:, :HIDDEN]
    t_ref[...] = y[:, HIDDEN:2 * HIDDEN]
    m_ref[...] = y[:, 2 * HIDDEN:]


@functools.partial(
    pl.kernel,
    mesh=plsc.VectorSubcoreMesh(core_axis_name="c", subcore_axis_name="s"),
    out_type=jax.ShapeDtypeStruct((N_NODES, HIDDEN), jnp.float32),
    scratch_types=[
        pltpu.VMEM((CHUNK,), jnp.int32),
        pltpu.VMEM((CHUNK, HIDDEN), jnp.float32),
        pltpu.SemaphoreType.DMA,
    ],
)
def _sc_gather(idx_hbm, table_hbm, out_hbm, idx_v, rows_v, sem):
    wid = lax.axis_index("s") * NC + lax.axis_index("c")
    for r in range(MAX_ROUNDS):
        c = wid + r * NW

        @pl.when(c < TOTAL_CHUNKS)
        def _():
            base = c * CHUNK
            pltpu.sync_copy(idx_hbm.at[pl.ds(base, CHUNK)], idx_v)
            pltpu.async_copy(table_hbm.at[idx_v], rows_v, sem).wait()
            pltpu.sync_copy(rows_v, out_hbm.at[pl.ds(base, CHUNK)])


def kernel(atomic_numbers, radial_basis, emb_table, W, b):
    idx = atomic_numbers.astype(jnp.int32) - 1
    h = _sc_gather(idx, emb_table)

    b2 = b.reshape(1, HIDDEN * 3)
    grid_e = N_EDGES // EDGE_BLOCK
    out_block = pl.BlockSpec((EDGE_BLOCK, HIDDEN), lambda i: (i, 0))
    s, t, m = pl.pallas_call(
        _proj_kernel,
        grid=(grid_e,),
        in_specs=[
            pl.BlockSpec((EDGE_BLOCK, NUM_RADIAL), lambda i: (i, 0)),
            pl.BlockSpec((NUM_RADIAL, HIDDEN * 3), lambda i: (0, 0)),
            pl.BlockSpec((1, HIDDEN * 3), lambda i: (0, 0)),
        ],
        out_specs=[out_block, out_block, out_block],
        out_shape=[jax.ShapeDtypeStruct((N_EDGES, HIDDEN), jnp.float32)] * 3,
    )(radial_basis, W, b2)

    return (h, m, s, t)


# speedup vs baseline: 1.5662x; 1.0109x over previous
import functools

import jax
import jax.numpy as jnp
from jax import lax
from jax.experimental import pallas as pl
from jax.experimental.pallas import tpu as pltpu
from jax.experimental.pallas import tpu_sc as plsc

N_NODES = 10000
N_EDGES = 320000
NUM_ELEMENTS = 100
HIDDEN = 128
NUM_RADIAL = 16

EDGE_BLOCK = 16000

NC = 2
NS = 16
NW = NC * NS
CHUNK = 80
TOTAL_CHUNKS = N_NODES // CHUNK
MAX_ROUNDS = -(-TOTAL_CHUNKS // NW)


def _proj_kernel(rb_ref, w_ref, b_ref, s_ref, t_ref, m_ref):
    y = jnp.dot(rb_ref[...], w_ref[...], preferred_element_type=jnp.float32)
    y = y + b_ref[...]
    s_ref[...] = y[:, :HIDDEN]
    t_ref[...] = y[:, HIDDEN:2 * HIDDEN]
    m_ref[...] = y[:, 2 * HIDDEN:]


@functools.partial(
    pl.kernel,
    mesh=plsc.VectorSubcoreMesh(core_axis_name="c", subcore_axis_name="s"),
    out_type=jax.ShapeDtypeStruct((N_NODES, HIDDEN), jnp.float32),
    scratch_types=[
        pltpu.VMEM((CHUNK,), jnp.int32),
        pltpu.VMEM((CHUNK, HIDDEN), jnp.float32),
        pltpu.SemaphoreType.DMA,
    ],
)
def _sc_gather(idx_hbm, table_hbm, out_hbm, idx_v, rows_v, sem):
    wid = lax.axis_index("s") * NC + lax.axis_index("c")
    for r in range(MAX_ROUNDS):
        c = wid + r * NW

        @pl.when(c < TOTAL_CHUNKS)
        def _():
            base = c * CHUNK
            pltpu.sync_copy(idx_hbm.at[pl.ds(base, CHUNK)], idx_v)
            pltpu.async_copy(table_hbm.at[idx_v], rows_v, sem).wait()
            pltpu.sync_copy(rows_v, out_hbm.at[pl.ds(base, CHUNK)])


def kernel(atomic_numbers, radial_basis, emb_table, W, b):
    idx = atomic_numbers.astype(jnp.int32) - 1
    h = _sc_gather(idx, emb_table)

    b2 = b.reshape(1, HIDDEN * 3)
    grid_e = N_EDGES // EDGE_BLOCK
    out_block = pl.BlockSpec((EDGE_BLOCK, HIDDEN), lambda i: (i, 0))
    s, t, m = pl.pallas_call(
        _proj_kernel,
        grid=(grid_e,),
        in_specs=[
            pl.BlockSpec((EDGE_BLOCK, NUM_RADIAL), lambda i: (i, 0)),
            pl.BlockSpec((NUM_RADIAL, HIDDEN * 3), lambda i: (0, 0)),
            pl.BlockSpec((1, HIDDEN * 3), lambda i: (0, 0)),
        ],
        out_specs=[out_block, out_block, out_block],
        out_shape=[jax.ShapeDtypeStruct((N_EDGES, HIDDEN), jnp.float32)] * 3,
        compiler_params=pltpu.CompilerParams(
            vmem_limit_bytes=100 * 1024 * 1024),
    )(radial_basis, W, b2)

    return (h, m, s, t)
